# baseline (speedup 1.0000x reference)
import functools

import jax
import jax.numpy as jnp
from jax import lax
from jax.experimental import pallas as pl
from jax.experimental.pallas import tpu as pltpu
from jax.experimental.pallas import tpu_sc as plsc

_NPOINTS = [1024, 512, 256, 128]
_RADII = [0.05, 0.1, 0.2, 0.3]
_NSAMPLE = 32
_INTERPRET = False


def _pc(*args, **kwargs):
    return pl.pallas_call(*args, interpret=_INTERPRET, **kwargs)


def _fps_body(xyz_ref, out_ref):
    B = xyz_ref.shape[1]
    x = xyz_ref[0]
    y = xyz_ref[1]
    z = xyz_ref[2]
    off = 0
    for npoint in _NPOINTS:
        N = x.shape[1]
        iota_n = lax.broadcasted_iota(jnp.int32, (B, N), 1)
        iota_p = lax.broadcasted_iota(jnp.int32, (B, npoint), 1)

        def body(i, st, x=x, y=y, z=z, iota_n=iota_n, iota_p=iota_p):
            d, far, sx, sy, sz = st
            oh = (iota_n == far).astype(jnp.float32)
            cx = jnp.sum(x * oh, axis=1, keepdims=True)
            cy = jnp.sum(y * oh, axis=1, keepdims=True)
            cz = jnp.sum(z * oh, axis=1, keepdims=True)
            hit = iota_p == i
            sx = jnp.where(hit, cx, sx)
            sy = jnp.where(hit, cy, sy)
            sz = jnp.where(hit, cz, sz)
            dist = (x - cx) ** 2 + (y - cy) ** 2 + (z - cz) ** 2
            d = jnp.minimum(d, dist)
            dmax = jnp.max(d, axis=1, keepdims=True)
            far = jnp.min(jnp.where(d == dmax, iota_n, N), axis=1,
                          keepdims=True)
            return d, far, sx, sy, sz

        d0 = jnp.full((B, N), 1e10, jnp.float32)
        far0 = jnp.zeros((B, 1), jnp.int32)
        s0 = jnp.zeros((B, npoint), jnp.float32)
        _, _, sx, sy, sz = lax.fori_loop(0, npoint, body,
                                         (d0, far0, s0, s0, s0))
        out_ref[0, :, off:off + npoint] = sx
        out_ref[1, :, off:off + npoint] = sy
        out_ref[2, :, off:off + npoint] = sz
        x, y, z = sx, sy, sz
        off += npoint


def _fps_all(xyz_planes):
    B = xyz_planes.shape[1]
    total = sum(_NPOINTS)
    return _pc(
        _fps_body,
        out_shape=jax.ShapeDtypeStruct((3, B, total), jnp.float32),
    )(xyz_planes)


def _bq_body(r2, nxc_ref, xp_ref, out_ref):
    S = nxc_ref.shape[1]
    N = xp_ref.shape[2]
    d2 = jnp.zeros((S, N), jnp.float32)
    for c in range(3):
        a = nxc_ref[0, :, c:c + 1]
        b = xp_ref[0, c:c + 1, :]
        d2 = d2 + (a - b) ** 2
    iota = lax.broadcasted_iota(jnp.int32, (S, N), 1)
    sel = jnp.where(d2 <= r2, iota, N)
    c0 = jnp.min(sel, axis=1, keepdims=True)
    first = jnp.where(c0 == N, 0, c0)
    out_ref[0, :, 0:1] = first
    sel = jnp.where(sel == c0, N, sel)
    for k in range(1, _NSAMPLE):
        ck = jnp.min(sel, axis=1, keepdims=True)
        out_ref[0, :, k:k + 1] = jnp.where(ck == N, first, ck)
        sel = jnp.where(sel == ck, N, sel)


def _ball_query(radius, new_xyz_c, xyz_p):
    B, S, _ = new_xyz_c.shape
    N = xyz_p.shape[2]
    return _pc(
        functools.partial(_bq_body, radius * radius),
        grid=(B,),
        in_specs=[
            pl.BlockSpec((1, S, 3), lambda b: (b, 0, 0)),
            pl.BlockSpec((1, 3, N), lambda b: (b, 0, 0)),
        ],
        out_specs=pl.BlockSpec((1, S, _NSAMPLE), lambda b: (b, 0, 0)),
        out_shape=jax.ShapeDtypeStruct((B, S, _NSAMPLE), jnp.int32),
    )(new_xyz_c, xyz_p)


def _chunk_size(b_per_w, row_bytes):
    budget = max(8, (380 * 1024) // row_bytes)
    best = 8
    c = 8
    while c <= b_per_w:
        if b_per_w % c == 0 and c <= budget:
            best = max(best, c)
        c += 8
    return best


def _sc_gather(table, idx):
    info = plsc.get_sparse_core_info()
    nc, ns = info.num_cores, info.num_subcores
    nw = nc * ns
    M = idx.shape[0]
    D = table.shape[1]
    dt = table.dtype
    b_per_w = M // nw
    chunk = _chunk_size(b_per_w, D * dt.itemsize)
    steps = b_per_w // chunk
    mesh = plsc.VectorSubcoreMesh(core_axis_name="c", subcore_axis_name="s")

    @functools.partial(
        pl.kernel,
        mesh=mesh,
        out_type=jax.ShapeDtypeStruct((M, D), dt),
        scratch_types=[
            pltpu.VMEM((chunk,), jnp.int32),
            pltpu.VMEM((chunk, D), dt),
            pltpu.SemaphoreType.DMA,
        ],
    )
    def k(table_hbm, idx_hbm, out_hbm, idx_v, rows_v, sem):
        wid = lax.axis_index("s") * nc + lax.axis_index("c")
        base = wid * b_per_w

        def step(t, carry):
            off = base + t * chunk
            pltpu.sync_copy(idx_hbm.at[pl.ds(off, chunk)], idx_v)
            pltpu.async_copy(table_hbm.at[idx_v], rows_v, sem).wait()
            pltpu.sync_copy(rows_v, out_hbm.at[pl.ds(off, chunk)])
            return carry

        lax.fori_loop(0, steps, step, 0)

    return k(table, idx)


def _sa_mlp_body(nlayers, g_ref, c_ref, *wb_out):
    ws = wb_out[:nlayers]
    bs = wb_out[nlayers:2 * nlayers]
    out_ref = wb_out[2 * nlayers]
    RB, Cin = g_ref.shape
    G = RB // _NSAMPLE
    x = g_ref[...].astype(jnp.float32) - c_ref[...]
    for li in range(nlayers):
        w = ws[li][...]
        b = bs[li][...]
        x = jnp.dot(x, w, preferred_element_type=jnp.float32) + b[0:1, :]
        x = jnp.maximum(x, 0.0)
    Cout = x.shape[1]
    x3 = x.reshape(G, _NSAMPLE, Cout)
    m = x3[:, 0, :]
    for j in range(1, _NSAMPLE):
        m = jnp.maximum(m, x3[:, j, :])
    out_ref[...] = m


def _sa_mlp(g, centers_exp, layers, rb):
    M, Cin = g.shape
    nlayers = len(layers)
    cout = layers[-1][0].shape[1]
    in_specs = [
        pl.BlockSpec((rb, Cin), lambda m: (m, 0)),
        pl.BlockSpec((rb, Cin), lambda m: (m, 0)),
    ]
    args = [g, centers_exp]
    for w, _ in layers:
        in_specs.append(pl.BlockSpec(w.shape, lambda m: (0, 0)))
        args.append(w)
    for _, b in layers:
        in_specs.append(pl.BlockSpec(b.shape, lambda m: (0, 0)))
        args.append(b)
    return _pc(
        functools.partial(_sa_mlp_body, nlayers),
        grid=(M // rb,),
        in_specs=in_specs,
        out_specs=pl.BlockSpec((rb // _NSAMPLE, cout), lambda m: (m, 0)),
        out_shape=jax.ShapeDtypeStruct((M // _NSAMPLE, cout), jnp.float32),
    )(*args)


def _knn_body(u_ref, kp_ref, idx_ref, w_ref):
    P = u_ref.shape[1]
    N = kp_ref.shape[2]
    d2 = jnp.zeros((P, N), jnp.float32)
    for c in range(3):
        a = u_ref[0, :, c:c + 1]
        b = kp_ref[0, c:c + 1, :]
        d2 = d2 + (a - b) ** 2
    iota = lax.broadcasted_iota(jnp.int32, (P, N), 1)
    recips = []
    for j in range(3):
        m = jnp.min(d2, axis=1, keepdims=True)
        i = jnp.min(jnp.where(d2 == m, iota, N), axis=1, keepdims=True)
        idx_ref[0, :, j:j + 1] = i
        recips.append(1.0 / (m + 1e-8))
        d2 = jnp.where(iota == i, jnp.float32(1e30), d2)
    norm = recips[0] + recips[1] + recips[2]
    for j in range(3):
        w_ref[0, :, j:j + 1] = recips[j] / norm


def _knn(u_c, known_p):
    B, P, _ = u_c.shape
    N = known_p.shape[2]
    return _pc(
        _knn_body,
        grid=(B,),
        in_specs=[
            pl.BlockSpec((1, P, 3), lambda b: (b, 0, 0)),
            pl.BlockSpec((1, 3, N), lambda b: (b, 0, 0)),
        ],
        out_specs=[
            pl.BlockSpec((1, P, 3), lambda b: (b, 0, 0)),
            pl.BlockSpec((1, P, 3), lambda b: (b, 0, 0)),
        ],
        out_shape=[
            jax.ShapeDtypeStruct((B, P, 3), jnp.int32),
            jax.ShapeDtypeStruct((B, P, 3), jnp.float32),
        ],
    )(u_c, known_p)


def _fp_body(g_ref, w_ref, f1_ref, wt_ref, b_ref, out_ref):
    RB3, C = g_ref.shape
    RB = RB3 // 3
    g3 = g_ref[...].astype(jnp.float32).reshape(RB, 3, C)
    w = w_ref[...]
    interp = g3[:, 0, :] * w[:, 0:1]
    interp = interp + g3[:, 1, :] * w[:, 1:2]
    interp = interp + g3[:, 2, :] * w[:, 2:3]
    x = jnp.concatenate([interp, f1_ref[...]], axis=1)
    x = jnp.dot(x, wt_ref[...], preferred_element_type=jnp.float32)
    out_ref[...] = jnp.maximum(x + b_ref[0:1, :], 0.0)


def _fp_mlp(g, w, f1, wt, b, rb):
    M = w.shape[0]
    C = g.shape[1]
    cout = wt.shape[1]
    return _pc(
        _fp_body,
        grid=(M // rb,),
        in_specs=[
            pl.BlockSpec((3 * rb, C), lambda m: (m, 0)),
            pl.BlockSpec((rb, 3), lambda m: (m, 0)),
            pl.BlockSpec((rb, f1.shape[1]), lambda m: (m, 0)),
            pl.BlockSpec(wt.shape, lambda m: (0, 0)),
            pl.BlockSpec(b.shape, lambda m: (0, 0)),
        ],
        out_specs=pl.BlockSpec((rb, cout), lambda m: (m, 0)),
        out_shape=jax.ShapeDtypeStruct((M, cout), jnp.float32),
    )(g, w, f1, wt, b)


def _head_body(x_ref, *refs):
    ws = refs[:6]
    bs = refs[6:12]
    out0, out1 = refs[12], refs[13]
    x = x_ref[...]

    def mlp(x, wlist, blist, last_act):
        n = len(wlist)
        for li in range(n):
            x = jnp.dot(x, wlist[li][...],
                        preferred_element_type=jnp.float32) + blist[li][0:1]
            if last_act or li < n - 1:
                x = jnp.maximum(x, 0.0)
        return x

    r0 = mlp(x, ws[0:2], bs[0:2], True)
    r1 = mlp(x, ws[2:4], bs[2:4], True)
    out0[...] = mlp(r0, ws[4:6], bs[4:6], False)
    out1[...] = mlp(r1, ws[4:6], bs[4:6], False)


def _head(x, ws, bs, rb):
    M, Cin = x.shape
    cout = ws[-1].shape[1]
    in_specs = [pl.BlockSpec((rb, Cin), lambda m: (m, 0))]
    for w in ws:
        in_specs.append(pl.BlockSpec(w.shape, lambda m: (0, 0)))
    for b in bs:
        in_specs.append(pl.BlockSpec(b.shape, lambda m: (0, 0)))
    return _pc(
        _head_body,
        grid=(M // rb,),
        in_specs=in_specs,
        out_specs=[pl.BlockSpec((rb, cout), lambda m: (m, 0))] * 2,
        out_shape=[jax.ShapeDtypeStruct((M, cout), jnp.float32)] * 2,
    )(x, *ws, *bs)


def _pad16(n):
    return (n + 15) // 16 * 16


def _pad128(n):
    return (n + 127) // 128 * 128


def _pad_first_w(w, d):
    wt = jnp.zeros((d, w.shape[0]), jnp.float32)
    return wt.at[:w.shape[1], :].set(w.T)


def kernel(points, params):
    B, N, _ = points.shape
    xyz = points[..., :3]
    xyz_planes = jnp.transpose(xyz, (2, 0, 1))
    fps_out = _fps_all(xyz_planes)

    l_planes = [xyz_planes]
    off = 0
    for npoint in _NPOINTS:
        l_planes.append(fps_out[:, :, off:off + npoint])
        off += npoint

    sa_rb = [2048, 2048, 1024, 512]
    feats_rows = None
    l_feats_rows = [None]
    for k in range(4):
        S = _NPOINTS[k]
        Nk = l_planes[k].shape[2]
        new_c = jnp.transpose(l_planes[k + 1], (1, 2, 0))
        idx = _ball_query(_RADII[k], new_c,
                          jnp.transpose(l_planes[k], (1, 0, 2)))
        xyz_rows = jnp.transpose(l_planes[k], (1, 2, 0)).reshape(B * Nk, 3)
        cin = 3 if feats_rows is None else 3 + feats_rows.shape[1]
        d = _pad128(cin)
        table = jnp.zeros((B * Nk, d), jnp.float32)
        table = table.at[:, :3].set(xyz_rows)
        if feats_rows is not None:
            table = table.at[:, 3:cin].set(feats_rows)
        gidx = (idx + (jnp.arange(B, dtype=jnp.int32) * Nk)[:, None, None])
        g = _sc_gather(table, gidx.reshape(-1))
        centers = jnp.zeros((B * S, d), jnp.float32)
        centers = centers.at[:, :3].set(new_c.reshape(B * S, 3))
        centers = jnp.repeat(centers, _NSAMPLE, axis=0)
        layers = []
        for li, (w, b) in enumerate(params['sa'][k]):
            wt = _pad_first_w(w, d) if li == 0 else w.T
            layers.append((wt, b[None, :]))
        feats_rows = _sa_mlp(g, centers, layers, sa_rb[k])
        l_feats_rows.append(feats_rows)

    u_c = xyz
    f1 = l_feats_rows[1]
    ups = []
    for kk in range(3):
        lev = kk + 2
        Nk = l_planes[lev].shape[2]
        idx, w = _knn(u_c, jnp.transpose(l_planes[lev], (1, 0, 2)))
        gidx = (idx + (jnp.arange(B, dtype=jnp.int32) * Nk)[:, None, None])
        g = _sc_gather(l_feats_rows[lev], gidx.reshape(-1))
        wfc, bfc = params['fp'][kk][0]
        up = _fp_mlp(g, w.reshape(B * N, 3), f1, wfc.T, bfc[None, :], 1024)
        ups.append(up)

    xcat = jnp.concatenate([xyz.reshape(B * N, 3), f1] + ups, axis=1)
    cin = xcat.shape[1]
    d = _pad16(cin)
    x = jnp.zeros((B * N, d), jnp.float32).at[:, :cin].set(xcat)
    ws, bs = [], []
    for grp in (params['fc'][0], params['fc'][1]):
        for li, (w, b) in enumerate(grp):
            ws.append(_pad_first_w(w, d) if li == 0 else w.T)
            bs.append(b[None, :])
    for li, (w, b) in enumerate(params['pcd']):
        if li == len(params['pcd']) - 1:
            wt = jnp.zeros((w.shape[1], 16), jnp.float32).at[:, :3].set(w.T)
            bp = jnp.zeros((1, 16), jnp.float32).at[:, :3].set(b[None, :])
            ws.append(wt)
            bs.append(bp)
        else:
            ws.append(w.T)
            bs.append(b[None, :])
    out0, out1 = _head(x, ws, bs, 1024)
    p0 = out0[:, :3].reshape(B, N, 3)
    p1 = out1[:, :3].reshape(B, N, 3)
    return jnp.concatenate([p0, p1], axis=1)


# speedup vs baseline: 1.0545x; 1.0545x over previous
import functools

import jax
import jax.numpy as jnp
from jax import lax
from jax.experimental import pallas as pl
from jax.experimental.pallas import tpu as pltpu
from jax.experimental.pallas import tpu_sc as plsc

_NPOINTS = [1024, 512, 256, 128]
_RADII = [0.05, 0.1, 0.2, 0.3]
_NSAMPLE = 32


def _pc(*args, **kwargs):
    return pl.pallas_call(*args, **kwargs)


def _fps_body(xyz_ref, out_ref):
    B = xyz_ref.shape[1]
    x = xyz_ref[0]
    y = xyz_ref[1]
    z = xyz_ref[2]
    off = 0
    for npoint in _NPOINTS:
        N = x.shape[1]
        iota_n = lax.broadcasted_iota(jnp.int32, (B, N), 1)
        iota_p = lax.broadcasted_iota(jnp.int32, (B, npoint), 1)

        def body(i, st, x=x, y=y, z=z, iota_n=iota_n, iota_p=iota_p):
            d, far, sx, sy, sz = st
            oh = (iota_n == far).astype(jnp.float32)
            cx = jnp.sum(x * oh, axis=1, keepdims=True)
            cy = jnp.sum(y * oh, axis=1, keepdims=True)
            cz = jnp.sum(z * oh, axis=1, keepdims=True)
            hit = iota_p == i
            sx = jnp.where(hit, cx, sx)
            sy = jnp.where(hit, cy, sy)
            sz = jnp.where(hit, cz, sz)
            dist = (x - cx) ** 2 + (y - cy) ** 2 + (z - cz) ** 2
            d = jnp.minimum(d, dist)
            dmax = jnp.max(d, axis=1, keepdims=True)
            far = jnp.min(jnp.where(d == dmax, iota_n, N), axis=1,
                          keepdims=True)
            return d, far, sx, sy, sz

        d0 = jnp.full((B, N), 1e10, jnp.float32)
        far0 = jnp.zeros((B, 1), jnp.int32)
        s0 = jnp.zeros((B, npoint), jnp.float32)
        _, _, sx, sy, sz = lax.fori_loop(0, npoint, body,
                                         (d0, far0, s0, s0, s0))
        out_ref[0, :, off:off + npoint] = sx
        out_ref[1, :, off:off + npoint] = sy
        out_ref[2, :, off:off + npoint] = sz
        x, y, z = sx, sy, sz
        off += npoint


def _fps_all(xyz_planes):
    B = xyz_planes.shape[1]
    total = sum(_NPOINTS)
    return _pc(
        _fps_body,
        out_shape=jax.ShapeDtypeStruct((3, B, total), jnp.float32),
    )(xyz_planes)


def _bq_body(r2, nxc_ref, xp_ref, out_ref):
    S = nxc_ref.shape[1]
    N = xp_ref.shape[2]
    d2 = jnp.zeros((S, N), jnp.float32)
    for c in range(3):
        a = nxc_ref[0, :, c:c + 1]
        b = xp_ref[0, c:c + 1, :]
        d2 = d2 + (a - b) ** 2
    iota = lax.broadcasted_iota(jnp.int32, (S, N), 1)
    sel = jnp.where(d2 <= r2, iota, N)
    c0 = jnp.min(sel, axis=1, keepdims=True)
    first = jnp.where(c0 == N, 0, c0)
    out_ref[0, :, 0:1] = first
    sel = jnp.where(sel == c0, N, sel)
    for k in range(1, _NSAMPLE):
        ck = jnp.min(sel, axis=1, keepdims=True)
        out_ref[0, :, k:k + 1] = jnp.where(ck == N, first, ck)
        sel = jnp.where(sel == ck, N, sel)


def _ball_query(radius, new_xyz_c, xyz_p):
    B, S, _ = new_xyz_c.shape
    N = xyz_p.shape[2]
    return _pc(
        functools.partial(_bq_body, radius * radius),
        grid=(B,),
        in_specs=[
            pl.BlockSpec((1, S, 3), lambda b: (b, 0, 0)),
            pl.BlockSpec((1, 3, N), lambda b: (b, 0, 0)),
        ],
        out_specs=pl.BlockSpec((1, S, _NSAMPLE), lambda b: (b, 0, 0)),
        out_shape=jax.ShapeDtypeStruct((B, S, _NSAMPLE), jnp.int32),
    )(new_xyz_c, xyz_p)


def _chunk_size(b_per_w, row_bytes):
    budget = max(8, (380 * 1024) // row_bytes)
    best = 8
    c = 8
    while c <= b_per_w:
        if b_per_w % c == 0 and c <= budget:
            best = max(best, c)
        c += 8
    return best


def _sc_gather(table, idx):
    info = plsc.get_sparse_core_info()
    nc, ns = info.num_cores, info.num_subcores
    nw = nc * ns
    M = idx.shape[0]
    D = table.shape[1]
    dt = table.dtype
    b_per_w = M // nw
    chunk = _chunk_size(b_per_w, D * dt.itemsize)
    steps = b_per_w // chunk
    mesh = plsc.VectorSubcoreMesh(core_axis_name="c", subcore_axis_name="s")

    @functools.partial(
        pl.kernel,
        mesh=mesh,
        out_type=jax.ShapeDtypeStruct((M, D), dt),
        scratch_types=[
            pltpu.VMEM((chunk,), jnp.int32),
            pltpu.VMEM((chunk, D), dt),
            pltpu.SemaphoreType.DMA,
        ],
    )
    def k(table_hbm, idx_hbm, out_hbm, idx_v, rows_v, sem):
        wid = lax.axis_index("s") * nc + lax.axis_index("c")
        base = wid * b_per_w

        def step(t, carry):
            off = base + t * chunk
            pltpu.sync_copy(idx_hbm.at[pl.ds(off, chunk)], idx_v)
            pltpu.async_copy(table_hbm.at[idx_v], rows_v, sem).wait()
            pltpu.sync_copy(rows_v, out_hbm.at[pl.ds(off, chunk)])
            return carry

        lax.fori_loop(0, steps, step, 0)

    return k(table, idx)


def _sc_gather1(table1d, idx):
    info = plsc.get_sparse_core_info()
    nc, ns = info.num_cores, info.num_subcores
    nw = nc * ns
    M = idx.shape[0]
    b_per_w = M // nw
    chunk = _chunk_size(b_per_w, 8)
    steps = b_per_w // chunk
    mesh = plsc.VectorSubcoreMesh(core_axis_name="c", subcore_axis_name="s")

    @functools.partial(
        pl.kernel,
        mesh=mesh,
        out_type=jax.ShapeDtypeStruct((M,), jnp.float32),
        scratch_types=[
            pltpu.VMEM((chunk,), jnp.int32),
            pltpu.VMEM((chunk,), jnp.float32),
            pltpu.SemaphoreType.DMA,
        ],
    )
    def k(table_hbm, idx_hbm, out_hbm, idx_v, rows_v, sem):
        wid = lax.axis_index("s") * nc + lax.axis_index("c")
        base = wid * b_per_w

        def step(t, carry):
            off = base + t * chunk
            pltpu.sync_copy(idx_hbm.at[pl.ds(off, chunk)], idx_v)
            pltpu.async_copy(table_hbm.at[idx_v], rows_v, sem).wait()
            pltpu.sync_copy(rows_v, out_hbm.at[pl.ds(off, chunk)])
            return carry

        lax.fori_loop(0, steps, step, 0)

    return k(table1d, idx)


def _sa_mlp_body(nlayers, g_ref, c_ref, *wb_out):
    ws = wb_out[:nlayers]
    bs = wb_out[nlayers:2 * nlayers]
    out_ref = wb_out[2 * nlayers]
    RB, Cin = g_ref.shape
    G = RB // _NSAMPLE
    x = g_ref[...].astype(jnp.float32) - c_ref[...]
    for li in range(nlayers):
        w = ws[li][...]
        b = bs[li][...]
        x = jnp.dot(x, w, preferred_element_type=jnp.float32) + b[0:1, :]
        x = jnp.maximum(x, 0.0)
    Cout = x.shape[1]
    x3 = x.reshape(G, _NSAMPLE, Cout)
    m = x3[:, 0, :]
    for j in range(1, _NSAMPLE):
        m = jnp.maximum(m, x3[:, j, :])
    out_ref[...] = m


def _sa_mlp(g, centers_exp, layers, rb):
    M, Cin = g.shape
    nlayers = len(layers)
    cout = layers[-1][0].shape[1]
    in_specs = [
        pl.BlockSpec((rb, Cin), lambda m: (m, 0)),
        pl.BlockSpec((rb, Cin), lambda m: (m, 0)),
    ]
    args = [g, centers_exp]
    for w, _ in layers:
        in_specs.append(pl.BlockSpec(w.shape, lambda m: (0, 0)))
        args.append(w)
    for _, b in layers:
        in_specs.append(pl.BlockSpec(b.shape, lambda m: (0, 0)))
        args.append(b)
    return _pc(
        functools.partial(_sa_mlp_body, nlayers),
        grid=(M // rb,),
        in_specs=in_specs,
        out_specs=pl.BlockSpec((rb // _NSAMPLE, cout), lambda m: (m, 0)),
        out_shape=jax.ShapeDtypeStruct((M // _NSAMPLE, cout), jnp.float32),
    )(*args)


def _sa1_body(nlayers, gp_ref, c_ref, *wb_out):
    ws = wb_out[:nlayers]
    bs = wb_out[nlayers:2 * nlayers]
    out_ref = wb_out[2 * nlayers]
    GB = gp_ref.shape[0]
    gp = gp_ref[...]
    c = c_ref[...]
    X = jnp.concatenate([gp[:, 4 * j:4 * j + 4] for j in range(_NSAMPLE)],
                        axis=0)
    X = X - jnp.concatenate([c] * _NSAMPLE, axis=0)
    x = X
    for li in range(nlayers):
        x = jnp.dot(x, ws[li][...],
                    preferred_element_type=jnp.float32) + bs[li][0:1, :]
        x = jnp.maximum(x, 0.0)
    m = x[0:GB]
    for j in range(1, _NSAMPLE):
        m = jnp.maximum(m, x[j * GB:(j + 1) * GB])
    out_ref[...] = m


def _sa1_mlp(gp, c4, layers, gb):
    M = gp.shape[0]
    nlayers = len(layers)
    cout = layers[-1][0].shape[1]
    in_specs = [
        pl.BlockSpec((gb, 128), lambda m: (m, 0)),
        pl.BlockSpec((gb, 4), lambda m: (m, 0)),
    ]
    args = [gp, c4]
    for w, _ in layers:
        in_specs.append(pl.BlockSpec(w.shape, lambda m: (0, 0)))
        args.append(w)
    for _, b in layers:
        in_specs.append(pl.BlockSpec(b.shape, lambda m: (0, 0)))
        args.append(b)
    return _pc(
        functools.partial(_sa1_body, nlayers),
        grid=(M // gb,),
        in_specs=in_specs,
        out_specs=pl.BlockSpec((gb, cout), lambda m: (m, 0)),
        out_shape=jax.ShapeDtypeStruct((M, cout), jnp.float32),
    )(*args)


def _knn_body(u_ref, kp_ref, idx_ref, w_ref):
    P = u_ref.shape[1]
    N = kp_ref.shape[2]
    d2 = jnp.zeros((P, N), jnp.float32)
    for c in range(3):
        a = u_ref[0, :, c:c + 1]
        b = kp_ref[0, c:c + 1, :]
        d2 = d2 + (a - b) ** 2
    iota = lax.broadcasted_iota(jnp.int32, (P, N), 1)
    recips = []
    for j in range(3):
        m = jnp.min(d2, axis=1, keepdims=True)
        i = jnp.min(jnp.where(d2 == m, iota, N), axis=1, keepdims=True)
        idx_ref[0, :, j:j + 1] = i
        recips.append(1.0 / (m + 1e-8))
        d2 = jnp.where(iota == i, jnp.float32(1e30), d2)
    norm = recips[0] + recips[1] + recips[2]
    for j in range(3):
        w_ref[0, :, j:j + 1] = recips[j] / norm


def _knn(u_c, known_p):
    B, P, _ = u_c.shape
    N = known_p.shape[2]
    return _pc(
        _knn_body,
        grid=(B,),
        in_specs=[
            pl.BlockSpec((1, P, 3), lambda b: (b, 0, 0)),
            pl.BlockSpec((1, 3, N), lambda b: (b, 0, 0)),
        ],
        out_specs=[
            pl.BlockSpec((1, P, 3), lambda b: (b, 0, 0)),
            pl.BlockSpec((1, P, 3), lambda b: (b, 0, 0)),
        ],
        out_shape=[
            jax.ShapeDtypeStruct((B, P, 3), jnp.int32),
            jax.ShapeDtypeStruct((B, P, 3), jnp.float32),
        ],
    )(u_c, known_p)


def _fp_body(g_ref, w_ref, f1_ref, wt_ref, b_ref, out_ref):
    RB3, C = g_ref.shape
    RB = RB3 // 3
    g3 = g_ref[...].astype(jnp.float32).reshape(RB, 3, C)
    w = w_ref[...]
    interp = g3[:, 0, :] * w[:, 0:1]
    interp = interp + g3[:, 1, :] * w[:, 1:2]
    interp = interp + g3[:, 2, :] * w[:, 2:3]
    x = jnp.concatenate([interp, f1_ref[...]], axis=1)
    x = jnp.dot(x, wt_ref[...], preferred_element_type=jnp.float32)
    out_ref[...] = jnp.maximum(x + b_ref[0:1, :], 0.0)


def _fp_mlp(g, w, f1, wt, b, rb):
    M = w.shape[0]
    C = g.shape[1]
    cout = wt.shape[1]
    return _pc(
        _fp_body,
        grid=(M // rb,),
        in_specs=[
            pl.BlockSpec((3 * rb, C), lambda m: (m, 0)),
            pl.BlockSpec((rb, 3), lambda m: (m, 0)),
            pl.BlockSpec((rb, f1.shape[1]), lambda m: (m, 0)),
            pl.BlockSpec(wt.shape, lambda m: (0, 0)),
            pl.BlockSpec(b.shape, lambda m: (0, 0)),
        ],
        out_specs=pl.BlockSpec((rb, cout), lambda m: (m, 0)),
        out_shape=jax.ShapeDtypeStruct((M, cout), jnp.float32),
    )(g, w, f1, wt, b)


def _head_body(x_ref, *refs):
    ws = refs[:6]
    bs = refs[6:12]
    out0, out1 = refs[12], refs[13]
    x = x_ref[...]

    def mlp(x, wlist, blist, last_act):
        n = len(wlist)
        for li in range(n):
            x = jnp.dot(x, wlist[li][...],
                        preferred_element_type=jnp.float32) + blist[li][0:1]
            if last_act or li < n - 1:
                x = jnp.maximum(x, 0.0)
        return x

    r0 = mlp(x, ws[0:2], bs[0:2], True)
    r1 = mlp(x, ws[2:4], bs[2:4], True)
    out0[...] = mlp(r0, ws[4:6], bs[4:6], False)
    out1[...] = mlp(r1, ws[4:6], bs[4:6], False)


def _head(x, ws, bs, rb):
    M, Cin = x.shape
    cout = ws[-1].shape[1]
    in_specs = [pl.BlockSpec((rb, Cin), lambda m: (m, 0))]
    for w in ws:
        in_specs.append(pl.BlockSpec(w.shape, lambda m: (0, 0)))
    for b in bs:
        in_specs.append(pl.BlockSpec(b.shape, lambda m: (0, 0)))
    return _pc(
        _head_body,
        grid=(M // rb,),
        in_specs=in_specs,
        out_specs=[pl.BlockSpec((rb, cout), lambda m: (m, 0))] * 2,
        out_shape=[jax.ShapeDtypeStruct((M, cout), jnp.float32)] * 2,
    )(x, *ws, *bs)


def _pad16(n):
    return (n + 15) // 16 * 16


def _pad128(n):
    return (n + 127) // 128 * 128


def _pad_first_w(w, d):
    wt = jnp.zeros((d, w.shape[0]), jnp.float32)
    return wt.at[:w.shape[1], :].set(w.T)


def kernel(points, params):
    B, N, _ = points.shape
    xyz = points[..., :3]
    xyz_planes = jnp.transpose(xyz, (2, 0, 1))
    fps_out = _fps_all(xyz_planes)

    l_planes = [xyz_planes]
    off = 0
    for npoint in _NPOINTS:
        l_planes.append(fps_out[:, :, off:off + npoint])
        off += npoint

    sa_rb = [2048, 2048, 1024, 512]
    feats_rows = None
    l_feats_rows = [None]
    for k in range(4):
        S = _NPOINTS[k]
        Nk = l_planes[k].shape[2]
        new_c = jnp.transpose(l_planes[k + 1], (1, 2, 0))
        idx = _ball_query(_RADII[k], new_c,
                          jnp.transpose(l_planes[k], (1, 0, 2)))
        xyz_rows = jnp.transpose(l_planes[k], (1, 2, 0)).reshape(B * Nk, 3)
        gidx = (idx + (jnp.arange(B, dtype=jnp.int32) * Nk)[:, None, None])
        if feats_rows is None:
            xyz4 = jnp.zeros((B * Nk, 4), jnp.float32).at[:, :3].set(
                xyz_rows)
            eidx = (gidx.reshape(-1)[:, None] * 4
                    + jnp.arange(4, dtype=jnp.int32)).reshape(-1)
            g1 = _sc_gather1(xyz4.reshape(-1), eidx)
            gp = g1.reshape(B * S, 4 * _NSAMPLE)
            c4 = jnp.zeros((B * S, 4), jnp.float32).at[:, :3].set(
                new_c.reshape(B * S, 3))
            layers = []
            for li, (w, b) in enumerate(params['sa'][k]):
                wt = _pad_first_w(w, 4) if li == 0 else w.T
                layers.append((wt, b[None, :]))
            feats_rows = _sa1_mlp(gp, c4, layers, 512)
        else:
            cin = 3 + feats_rows.shape[1]
            d = _pad128(cin)
            table = jnp.zeros((B * Nk, d), jnp.float32)
            table = table.at[:, :3].set(xyz_rows)
            table = table.at[:, 3:cin].set(feats_rows)
            g = _sc_gather(table, gidx.reshape(-1))
            centers = jnp.zeros((B * S, d), jnp.float32)
            centers = centers.at[:, :3].set(new_c.reshape(B * S, 3))
            centers = jnp.repeat(centers, _NSAMPLE, axis=0)
            layers = []
            for li, (w, b) in enumerate(params['sa'][k]):
                wt = _pad_first_w(w, d) if li == 0 else w.T
                layers.append((wt, b[None, :]))
            feats_rows = _sa_mlp(g, centers, layers, sa_rb[k])
        l_feats_rows.append(feats_rows)

    u_c = xyz
    f1 = l_feats_rows[1]
    ups = []
    for kk in range(3):
        lev = kk + 2
        Nk = l_planes[lev].shape[2]
        idx, w = _knn(u_c, jnp.transpose(l_planes[lev], (1, 0, 2)))
        gidx = (idx + (jnp.arange(B, dtype=jnp.int32) * Nk)[:, None, None])
        g = _sc_gather(l_feats_rows[lev], gidx.reshape(-1))
        wfc, bfc = params['fp'][kk][0]
        up = _fp_mlp(g, w.reshape(B * N, 3), f1, wfc.T, bfc[None, :], 1024)
        ups.append(up)

    xcat = jnp.concatenate([xyz.reshape(B * N, 3), f1] + ups, axis=1)
    cin = xcat.shape[1]
    d = _pad16(cin)
    x = jnp.zeros((B * N, d), jnp.float32).at[:, :cin].set(xcat)
    ws, bs = [], []
    for grp in (params['fc'][0], params['fc'][1]):
        for li, (w, b) in enumerate(grp):
            ws.append(_pad_first_w(w, d) if li == 0 else w.T)
            bs.append(b[None, :])
    for li, (w, b) in enumerate(params['pcd']):
        if li == len(params['pcd']) - 1:
            wt = jnp.zeros((w.shape[1], 16), jnp.float32).at[:, :3].set(w.T)
            bp = jnp.zeros((1, 16), jnp.float32).at[:, :3].set(b[None, :])
            ws.append(wt)
            bs.append(bp)
        else:
            ws.append(w.T)
            bs.append(b[None, :])
    out0, out1 = _head(x, ws, bs, 1024)
    p0 = out0[:, :3].reshape(B, N, 3)
    p1 = out1[:, :3].reshape(B, N, 3)
    return jnp.concatenate([p0, p1], axis=1)
